# baseline (device time: 58529 ns/iter reference)
import jax
import jax.numpy as jnp
from jax import lax
from jax.experimental import pallas as pl
from jax.experimental.pallas import tpu as pltpu

T = 256
D = 512
VP = 4096
V = 2 * VP
CHUNK = 256
NCH = VP // CHUNK
PRE = 4


def kernel(x, W):
    def body(x_ref, w_ref, out_ref, stats_ref, send_sems, recv_sems,
             st_send_sem, st_recv_sem):
        my_x = lax.axis_index("x")
        my_y = lax.axis_index("y")
        nbr = (1 - my_x, my_y)

        barrier_sem = pltpu.get_barrier_semaphore()
        pl.semaphore_signal(
            barrier_sem, inc=1, device_id=nbr,
            device_id_type=pl.DeviceIdType.MESH,
        )
        pl.semaphore_wait(barrier_sem, 1)

        xv = x_ref[...]

        def chunk_rdma(lo, c):
            return pltpu.make_async_remote_copy(
                src_ref=out_ref.at[:, pl.ds(lo, CHUNK)],
                dst_ref=out_ref.at[:, pl.ds(lo, CHUNK)],
                send_sem=send_sems.at[c],
                recv_sem=recv_sems.at[c],
                device_id=nbr,
                device_id_type=pl.DeviceIdType.MESH,
            )

        def run(my_lo):
            nbr_lo = VP - my_lo
            rdmas = []
            s_loc = jnp.zeros((T, 1), jnp.float32)
            for c in range(NCH):
                lo = my_lo + c * CHUNK
                logits_c = jnp.dot(
                    xv, w_ref[:, c * CHUNK:(c + 1) * CHUNK],
                    preferred_element_type=jnp.float32,
                )
                e_c = jnp.exp(logits_c)
                s_loc = s_loc + jnp.sum(e_c, axis=1, keepdims=True)
                out_ref[:, lo:lo + CHUNK] = e_c
                rdma = chunk_rdma(lo, c)
                rdmas.append(rdma)
                if c < PRE:
                    rdma.start()

            stats_ref[0] = jnp.broadcast_to(s_loc, (T, 8))
            st_rdma = pltpu.make_async_remote_copy(
                src_ref=stats_ref.at[0],
                dst_ref=stats_ref.at[1],
                send_sem=st_send_sem,
                recv_sem=st_recv_sem,
                device_id=nbr,
                device_id_type=pl.DeviceIdType.MESH,
            )
            st_rdma.start()
            for c in range(PRE, NCH):
                rdmas[c].start()

            st_rdma.wait_recv()
            inv = 1.0 / (s_loc + stats_ref[1, :, 0:1])
            for c in range(NCH):
                rdmas[c].wait_recv()
                lo = nbr_lo + c * CHUNK
                out_ref[:, lo:lo + CHUNK] = out_ref[:, lo:lo + CHUNK] * inv
                rdmas[c].wait_send()
                lo = my_lo + c * CHUNK
                out_ref[:, lo:lo + CHUNK] = out_ref[:, lo:lo + CHUNK] * inv
            st_rdma.wait_send()

        @pl.when(my_x == 0)
        def _():
            run(0)

        @pl.when(my_x == 1)
        def _():
            run(VP)

    return pl.pallas_call(
        body,
        out_shape=jax.ShapeDtypeStruct((T, V), jnp.float32),
        in_specs=[
            pl.BlockSpec(memory_space=pltpu.VMEM),
            pl.BlockSpec(memory_space=pltpu.VMEM),
        ],
        out_specs=pl.BlockSpec(memory_space=pltpu.VMEM),
        scratch_shapes=[
            pltpu.VMEM((2, T, 8), jnp.float32),
            pltpu.SemaphoreType.DMA((NCH,)),
            pltpu.SemaphoreType.DMA((NCH,)),
            pltpu.SemaphoreType.DMA,
            pltpu.SemaphoreType.DMA,
        ],
        compiler_params=pltpu.CompilerParams(collective_id=0),
    )(x, W)


# device time: 58432 ns/iter; 1.0017x vs baseline; 1.0017x over previous
import jax
import jax.numpy as jnp
from jax import lax
from jax.experimental import pallas as pl
from jax.experimental.pallas import tpu as pltpu

T = 256
D = 512
VP = 4096
V = 2 * VP
CHUNK = 512
NCH = VP // CHUNK


def kernel(x, W):
    def body(x_ref, w_ref, out_ref, stats_ref, send_sems, recv_sems,
             st_send_sem, st_recv_sem):
        my_x = lax.axis_index("x")
        my_y = lax.axis_index("y")
        nbr = (1 - my_x, my_y)

        barrier_sem = pltpu.get_barrier_semaphore()
        pl.semaphore_signal(
            barrier_sem, inc=1, device_id=nbr,
            device_id_type=pl.DeviceIdType.MESH,
        )
        pl.semaphore_wait(barrier_sem, 1)

        xv = x_ref[...]

        def run(my_lo):
            nbr_lo = VP - my_lo
            rdmas = []
            s_loc = jnp.zeros((T, 1), jnp.float32)
            for c in range(NCH):
                lo = my_lo + c * CHUNK
                logits_c = jnp.dot(
                    xv, w_ref[:, c * CHUNK:(c + 1) * CHUNK],
                    preferred_element_type=jnp.float32,
                )
                e_c = jnp.exp(logits_c)
                s_loc = s_loc + jnp.sum(e_c, axis=1, keepdims=True)
                out_ref[:, lo:lo + CHUNK] = e_c
                rdma = pltpu.make_async_remote_copy(
                    src_ref=out_ref.at[:, pl.ds(lo, CHUNK)],
                    dst_ref=out_ref.at[:, pl.ds(lo, CHUNK)],
                    send_sem=send_sems.at[c],
                    recv_sem=recv_sems.at[c],
                    device_id=nbr,
                    device_id_type=pl.DeviceIdType.MESH,
                )
                rdmas.append(rdma)
                if c < NCH - 1:
                    rdma.start()

            stats_ref[0] = jnp.broadcast_to(s_loc, (T, 8))
            st_rdma = pltpu.make_async_remote_copy(
                src_ref=stats_ref.at[0],
                dst_ref=stats_ref.at[1],
                send_sem=st_send_sem,
                recv_sem=st_recv_sem,
                device_id=nbr,
                device_id_type=pl.DeviceIdType.MESH,
            )
            st_rdma.start()
            rdmas[NCH - 1].start()

            st_rdma.wait_recv()
            inv = 1.0 / (s_loc + stats_ref[1, :, 0:1])
            for c in range(NCH):
                rdmas[c].wait_recv()
                lo = nbr_lo + c * CHUNK
                out_ref[:, lo:lo + CHUNK] = out_ref[:, lo:lo + CHUNK] * inv
                rdmas[c].wait_send()
                lo = my_lo + c * CHUNK
                out_ref[:, lo:lo + CHUNK] = out_ref[:, lo:lo + CHUNK] * inv
            st_rdma.wait_send()

        @pl.when(my_x == 0)
        def _():
            run(0)

        @pl.when(my_x == 1)
        def _():
            run(VP)

    return pl.pallas_call(
        body,
        out_shape=jax.ShapeDtypeStruct((T, V), jnp.float32),
        in_specs=[
            pl.BlockSpec(memory_space=pltpu.VMEM),
            pl.BlockSpec(memory_space=pltpu.VMEM),
        ],
        out_specs=pl.BlockSpec(memory_space=pltpu.VMEM),
        scratch_shapes=[
            pltpu.VMEM((2, T, 8), jnp.float32),
            pltpu.SemaphoreType.DMA((NCH,)),
            pltpu.SemaphoreType.DMA((NCH,)),
            pltpu.SemaphoreType.DMA,
            pltpu.SemaphoreType.DMA,
        ],
        compiler_params=pltpu.CompilerParams(collective_id=0),
    )(x, W)


# device time: 40424 ns/iter; 1.4479x vs baseline; 1.4455x over previous
import jax
import jax.numpy as jnp
from jax import lax
from jax.experimental import pallas as pl
from jax.experimental.pallas import tpu as pltpu

T = 256
D = 512
VP = 4096
V = 2 * VP
H = VP // 2
CS = 256
NX = H // CS


def kernel(x, W):
    def body(x_ref, w_ref, out_ref, xsend_sems, xrecv_sems,
             fsend_sems, frecv_sems):
        my_x = lax.axis_index("x")
        my_y = lax.axis_index("y")
        xnbr = (1 - my_x, my_y)
        ynbr = (my_x, 1 - my_y)

        bsem = pltpu.get_barrier_semaphore()
        for d in (xnbr, ynbr):
            pl.semaphore_signal(bsem, inc=1, device_id=d,
                                device_id_type=pl.DeviceIdType.MESH)
        pl.semaphore_wait(bsem, 2)

        xv = x_ref[...]
        my_lo = my_x * VP
        nbr_lo = (1 - my_x) * VP
        mine_half = my_y * H
        other_half = (1 - my_y) * H

        s_loc = jnp.zeros((T, 1), jnp.float32)
        xrdmas = []
        for c in range(NX):
            woff = mine_half + c * CS
            e = jnp.exp(jnp.dot(xv, w_ref[:, pl.ds(woff, CS)],
                                preferred_element_type=jnp.float32))
            s_loc = s_loc + jnp.sum(e, axis=1, keepdims=True)
            out_ref[:, pl.ds(my_lo + woff, CS)] = e
            rdma = pltpu.make_async_remote_copy(
                src_ref=out_ref.at[:, pl.ds(my_lo + woff, CS)],
                dst_ref=out_ref.at[:, pl.ds(my_lo + woff, CS)],
                send_sem=xsend_sems.at[c], recv_sem=xrecv_sems.at[c],
                device_id=xnbr, device_id_type=pl.DeviceIdType.MESH)
            rdma.start()
            xrdmas.append(rdma)
        for c in range(NX):
            woff = other_half + c * CS
            e = jnp.exp(jnp.dot(xv, w_ref[:, pl.ds(woff, CS)],
                                preferred_element_type=jnp.float32))
            s_loc = s_loc + jnp.sum(e, axis=1, keepdims=True)
            out_ref[:, pl.ds(my_lo + woff, CS)] = e

        s_nbr = jnp.zeros((T, 1), jnp.float32)
        frdmas = []
        for c in range(NX):
            xrdmas[c].wait_recv()
            lo = nbr_lo + mine_half + c * CS
            f = pltpu.make_async_remote_copy(
                src_ref=out_ref.at[:, pl.ds(lo, CS)],
                dst_ref=out_ref.at[:, pl.ds(lo, CS)],
                send_sem=fsend_sems.at[c], recv_sem=frecv_sems.at[c],
                device_id=ynbr, device_id_type=pl.DeviceIdType.MESH)
            f.start()
            frdmas.append(f)
            s_nbr = s_nbr + jnp.sum(out_ref[:, pl.ds(lo, CS)],
                                    axis=1, keepdims=True)
        for c in range(NX):
            lo = nbr_lo + other_half + c * CS
            recv = pltpu.make_async_remote_copy(
                src_ref=out_ref.at[:, pl.ds(lo, CS)],
                dst_ref=out_ref.at[:, pl.ds(lo, CS)],
                send_sem=fsend_sems.at[c], recv_sem=frecv_sems.at[c],
                device_id=ynbr, device_id_type=pl.DeviceIdType.MESH)
            recv.wait_recv()
            s_nbr = s_nbr + jnp.sum(out_ref[:, pl.ds(lo, CS)],
                                    axis=1, keepdims=True)
        for r in xrdmas:
            r.wait_send()
        for f in frdmas:
            f.wait_send()

        inv = 1.0 / (s_loc + s_nbr)
        out_ref[...] = out_ref[...] * inv

    return pl.pallas_call(
        body,
        out_shape=jax.ShapeDtypeStruct((T, V), jnp.float32),
        in_specs=[
            pl.BlockSpec(memory_space=pltpu.VMEM),
            pl.BlockSpec(memory_space=pltpu.VMEM),
        ],
        out_specs=pl.BlockSpec(memory_space=pltpu.VMEM),
        scratch_shapes=[
            pltpu.SemaphoreType.DMA((NX,)),
            pltpu.SemaphoreType.DMA((NX,)),
            pltpu.SemaphoreType.DMA((NX,)),
            pltpu.SemaphoreType.DMA((NX,)),
        ],
        compiler_params=pltpu.CompilerParams(collective_id=0),
    )(x, W)
